# Initial kernel scaffold; baseline (speedup 1.0000x reference)
import jax
import jax.numpy as jnp
from jax.experimental import pallas as pl


def kernel(x, W, conv_w, bn_gamma, bn_beta):
    raise NotImplementedError("write your pallas kernel here")


# speedup vs baseline: 1.7942x; 1.7942x over previous
import functools

import jax
import jax.numpy as jnp
from jax import lax
from jax.experimental import pallas as pl
from jax.experimental.pallas import tpu as pltpu


def _fused_kernel(x_ref, m_ref, w_ref, pmask_ref, bmask_ref, y_ref,
                  xpad_ref, hi_ref, oh_ref, fpad_ref,
                  *, block_num, width, chunk):
    C, P = x_ref.shape[1], x_ref.shape[2]
    F = y_ref.shape[1]
    K = block_num
    W = width
    f32 = jnp.float32
    bf16 = jnp.bfloat16
    pad = W + 1

    x = x_ref[0]

    xpad_ref[...] = jnp.zeros_like(xpad_ref)
    xpad_ref[:, pad:pad + P] = x
    ec = pmask_ref[0:1, :]
    er = pmask_ref[1:2, :]
    xl = xpad_ref[:, pad - 1:pad - 1 + P]
    xr = xpad_ref[:, pad + 1:pad + 1 + P]
    xu = xpad_ref[:, pad - W:pad - W + P]
    xd = xpad_ref[:, pad + W:pad + W + P]
    dpp = xpad_ref[:, pad + W + 1:pad + W + 1 + P]
    dpm = xpad_ref[:, pad + W - 1:pad + W - 1 + P]
    dmp = xpad_ref[:, pad - W + 1:pad - W + 1 + P]
    dmm = xpad_ref[:, pad - W - 1:pad - W - 1 + P]
    ph = ec * xr + (1.0 - ec) * xl
    pv = er * xd + (1.0 - er) * xu
    pd = (er * ec * dpp + er * (1.0 - ec) * dpm
          + (1.0 - er) * ec * dmp + (1.0 - er) * (1.0 - ec) * dmm)
    hi_ref[...] = x - 0.25 * (x + ph + pv + pd)

    mx = jnp.max(hi_ref[...], axis=0, keepdims=True)
    pref = jnp.full((1, P), -jnp.inf, f32)
    for c in range(K):
        hc = hi_ref[c:c + 1, :]
        oh_ref[c:c + 1, :] = jnp.where((hc == mx) & (hc > pref), 1.0, 0.0)
        pref = jnp.maximum(pref, hc)
    onehot = oh_ref[...]

    sums = lax.dot_general(onehot, x, (((1,), (1,)), ((), ())),
                           preferred_element_type=f32)
    counts = jnp.sum(onehot, axis=1, keepdims=True)
    means = sums / (counts + (counts == 0).astype(f32))
    M = m_ref[...]
    q = jnp.dot(means, M, preferred_element_type=f32)
    g = lax.dot_general(q, means, (((1,), (1,)), ((), ())),
                        preferred_element_type=f32)
    eye = (lax.broadcasted_iota(jnp.int32, (K, K), 0) ==
           lax.broadcasted_iota(jnp.int32, (K, K), 1)).astype(f32)
    diag_col = jnp.sum(g * eye, axis=1, keepdims=True)
    diag_row = jnp.sum(g * eye, axis=0, keepdims=True)
    quad = diag_col + diag_row - 2.0 * g
    adj = jnp.exp(-quad) * (1.0 - eye)
    adjm = jnp.dot(adj, means, preferred_element_type=f32)
    feat = x + lax.dot_general(adjm, onehot, (((0,), (0,)), ((), ())),
                               preferred_element_type=f32)

    fpad_ref[...] = jnp.zeros_like(fpad_ref)
    fpad_ref[:, pad:pad + P] = feat.astype(bf16)
    for off in range(0, P, chunk):
        acc = jnp.zeros((F, chunk), f32)
        t = 0
        for di in (-1, 0, 1):
            for dj in (-1, 0, 1):
                start = pad + di * W + dj + off
                s = fpad_ref[:, start:start + chunk]
                if dj == -1:
                    s = s * bmask_ref[0:1, off:off + chunk]
                elif dj == 1:
                    s = s * bmask_ref[1:2, off:off + chunk]
                acc = acc + jnp.dot(w_ref[t], s,
                                    preferred_element_type=f32)
                t += 1
        y_ref[0, :, off:off + chunk] = acc


def _bn_kernel(y_ref, g_ref, b_ref, o_ref):
    B, Fb, P = y_ref.shape
    n = B * P
    s = jnp.zeros((Fb, 1), jnp.float32)
    for b in range(B):
        s = s + jnp.sum(y_ref[b], axis=1, keepdims=True)
    mean = s * (1.0 / n)
    ss = jnp.zeros((Fb, 1), jnp.float32)
    for b in range(B):
        d = y_ref[b] - mean
        ss = ss + jnp.sum(d * d, axis=1, keepdims=True)
    inv = lax.rsqrt(ss * (1.0 / n) + 1e-5)
    scale = inv * g_ref[...]
    shift = b_ref[...] - mean * scale
    for b in range(B):
        o_ref[b] = y_ref[b] * scale + shift


def kernel(x, W, conv_w, bn_gamma, bn_beta):
    B, C, H, Wd = x.shape
    P = H * Wd
    K = 16
    F = conv_w.shape[0]
    f32 = jnp.float32
    bf16 = jnp.bfloat16
    chunk = 512 if P % 512 == 0 else P

    x_bcp = x.reshape(B, C, P).astype(f32)
    M = jnp.dot(W, W.T).astype(f32)
    w9 = conv_w.transpose(2, 3, 0, 1).reshape(9, F, C).astype(bf16)

    col = jnp.arange(P, dtype=jnp.int32) % Wd
    row = jnp.arange(P, dtype=jnp.int32) // Wd
    pmask = jnp.stack([(col % 2 == 0), (row % 2 == 0)]).astype(f32)
    bmask = jnp.stack([(col != 0), (col != Wd - 1)]).astype(bf16)

    kb = functools.partial(_fused_kernel, block_num=K, width=Wd, chunk=chunk)
    y = pl.pallas_call(
        kb,
        out_shape=jax.ShapeDtypeStruct((B, F, P), f32),
        grid=(B,),
        in_specs=[pl.BlockSpec((1, C, P), lambda b: (b, 0, 0)),
                  pl.BlockSpec((C, C), lambda b: (0, 0)),
                  pl.BlockSpec((9, F, C), lambda b: (0, 0, 0)),
                  pl.BlockSpec((2, P), lambda b: (0, 0)),
                  pl.BlockSpec((2, P), lambda b: (0, 0))],
        out_specs=pl.BlockSpec((1, F, P), lambda b: (b, 0, 0)),
        scratch_shapes=[pltpu.VMEM((C, P + 2 * Wd + 2), f32),
                        pltpu.VMEM((C, P), f32),
                        pltpu.VMEM((K, P), f32),
                        pltpu.VMEM((C, P + 2 * Wd + 2), bf16)],
        compiler_params=pltpu.CompilerParams(dimension_semantics=("parallel",)),
    )(x_bcp, M, w9, pmask, bmask)

    Fb = 16
    y_bn = pl.pallas_call(
        _bn_kernel,
        out_shape=jax.ShapeDtypeStruct((B, F, P), f32),
        grid=(F // Fb,),
        in_specs=[pl.BlockSpec((B, Fb, P), lambda f: (0, f, 0)),
                  pl.BlockSpec((Fb, 1), lambda f: (f, 0)),
                  pl.BlockSpec((Fb, 1), lambda f: (f, 0))],
        out_specs=pl.BlockSpec((B, Fb, P), lambda f: (0, f, 0)),
        compiler_params=pltpu.CompilerParams(dimension_semantics=("parallel",)),
    )(y, bn_gamma.reshape(F, 1).astype(f32), bn_beta.reshape(F, 1).astype(f32))

    return y_bn.reshape(B, F, H, Wd)


# speedup vs baseline: 1.8525x; 1.0325x over previous
import functools

import jax
import jax.numpy as jnp
from jax import lax
from jax.experimental import pallas as pl
from jax.experimental.pallas import tpu as pltpu


def _fused_kernel(x_ref, m_ref, w_ref, pmask_ref, bmask_ref,
                  y_ref, par_ref,
                  xpad_ref, hpad_ref, hi16_ref, oh_ref,
                  f0_ref, fl_ref, fr_ref,
                  *, block_num, width, chunk):
    C, P = x_ref.shape[1], x_ref.shape[2]
    F = y_ref.shape[1]
    K = block_num
    W = width
    f32 = jnp.float32
    bf16 = jnp.bfloat16
    pad = W

    x = x_ref[0]

    xpad_ref[:, 0:pad] = jnp.zeros((C, pad), f32)
    xpad_ref[:, pad + P:] = jnp.zeros((C, W), f32)
    xpad_ref[:, pad:pad + P] = x
    ec = pmask_ref[0:1, :]
    er = pmask_ref[1:2, :]
    xl = xpad_ref[:, pad - 1:pad - 1 + P]
    xr = xpad_ref[:, pad + 1:pad + 1 + P]
    ph = ec * xr + (1.0 - ec) * xl
    hpad_ref[:, 0:pad] = jnp.zeros((C, pad), f32)
    hpad_ref[:, pad + P:] = jnp.zeros((C, W), f32)
    hpad_ref[:, pad:pad + P] = ph
    xu = xpad_ref[:, 0:P]
    xd = xpad_ref[:, 2 * W:2 * W + P]
    pv = er * xd + (1.0 - er) * xu
    phu = hpad_ref[:, 0:P]
    phd = hpad_ref[:, 2 * W:2 * W + P]
    pd = er * phd + (1.0 - er) * phu
    hi = x - 0.25 * (x + ph + pv + pd)

    mx = jnp.max(hi, axis=0, keepdims=True)
    hi16_ref[...] = hi[0:K]
    pref = jnp.full((1, P), -jnp.inf, f32)
    for c in range(K):
        hc = hi16_ref[c:c + 1, :]
        oh_ref[c:c + 1, :] = jnp.where((hc == mx) & (hc > pref), 1.0, 0.0)
        pref = jnp.maximum(pref, hc)
    onehot = oh_ref[...]

    sums = lax.dot_general(onehot, x, (((1,), (1,)), ((), ())),
                           preferred_element_type=f32)
    counts = jnp.sum(onehot, axis=1, keepdims=True)
    means = sums / (counts + (counts == 0).astype(f32))
    M = m_ref[...]
    q = jnp.dot(means, M, preferred_element_type=f32)
    g = lax.dot_general(q, means, (((1,), (1,)), ((), ())),
                        preferred_element_type=f32)
    eye = (lax.broadcasted_iota(jnp.int32, (K, K), 0) ==
           lax.broadcasted_iota(jnp.int32, (K, K), 1)).astype(f32)
    diag_col = jnp.sum(g * eye, axis=1, keepdims=True)
    diag_row = jnp.sum(g * eye, axis=0, keepdims=True)
    quad = diag_col + diag_row - 2.0 * g
    adj = jnp.exp(-quad) * (1.0 - eye)
    adjm = jnp.dot(adj, means, preferred_element_type=f32)
    feat = x + lax.dot_general(adjm, onehot, (((0,), (0,)), ((), ())),
                               preferred_element_type=f32)

    zpadh = jnp.zeros((C, pad), bf16)
    f0_ref[:, 0:pad] = zpadh
    f0_ref[:, pad + P:] = zpadh
    f0_ref[:, pad:pad + P] = feat.astype(bf16)
    fl_ref[:, 0:pad] = zpadh
    fl_ref[:, pad + P:] = zpadh
    fl_ref[:, pad:pad + P] = (f0_ref[:, pad - 1:pad - 1 + P]
                              * bmask_ref[0:1, :])
    fr_ref[:, 0:pad] = zpadh
    fr_ref[:, pad + P:] = zpadh
    fr_ref[:, pad:pad + P] = (f0_ref[:, pad + 1:pad + 1 + P]
                              * bmask_ref[1:2, :])

    bsum = jnp.zeros((F, 1), f32)
    bsq = jnp.zeros((F, 1), f32)
    for off in range(0, P, chunk):
        acc = jnp.zeros((F, chunk), f32)
        t = 0
        for di in (-1, 0, 1):
            base = pad + di * W + off
            for fref in (fl_ref, f0_ref, fr_ref):
                s = fref[:, base:base + chunk]
                acc = acc + jnp.dot(w_ref[t], s,
                                    preferred_element_type=f32)
                t += 1
        y_ref[0, :, off:off + chunk] = acc.astype(bf16)
        bsum = bsum + jnp.sum(acc, axis=1, keepdims=True)
        bsq = bsq + jnp.sum(acc * acc, axis=1, keepdims=True)
    par_ref[0, :, 0:1] = bsum
    par_ref[0, :, 1:2] = bsq


def _bn_kernel(y_ref, par_ref, g_ref, b_ref, o_ref):
    B = par_ref.shape[0]
    Fb, P = y_ref.shape[1], y_ref.shape[2]
    n = B * P
    s = jnp.zeros((Fb, 1), jnp.float32)
    sq = jnp.zeros((Fb, 1), jnp.float32)
    for b in range(B):
        s = s + par_ref[b, :, 0:1]
        sq = sq + par_ref[b, :, 1:2]
    mean = s * (1.0 / n)
    var = sq * (1.0 / n) - mean * mean
    inv = lax.rsqrt(var + 1e-5)
    scale = inv * g_ref[...]
    shift = b_ref[...] - mean * scale
    o_ref[0] = y_ref[0].astype(jnp.float32) * scale + shift


def kernel(x, W, conv_w, bn_gamma, bn_beta):
    B, C, H, Wd = x.shape
    P = H * Wd
    K = 16
    F = conv_w.shape[0]
    f32 = jnp.float32
    bf16 = jnp.bfloat16
    chunk = 1024 if P % 1024 == 0 else P

    x_bcp = x.reshape(B, C, P).astype(f32)
    M = jnp.dot(W, W.T).astype(f32)
    w9 = conv_w.transpose(2, 3, 0, 1).reshape(9, F, C).astype(bf16)

    col = jnp.arange(P, dtype=jnp.int32) % Wd
    row = jnp.arange(P, dtype=jnp.int32) // Wd
    pmask = jnp.stack([(col % 2 == 0), (row % 2 == 0)]).astype(f32)
    bmask = jnp.stack([(col != 0), (col != Wd - 1)]).astype(bf16)

    kb = functools.partial(_fused_kernel, block_num=K, width=Wd, chunk=chunk)
    pe = P + 2 * Wd
    y, par = pl.pallas_call(
        kb,
        out_shape=[jax.ShapeDtypeStruct((B, F, P), bf16),
                   jax.ShapeDtypeStruct((B, F, 128), f32)],
        grid=(B,),
        in_specs=[pl.BlockSpec((1, C, P), lambda b: (b, 0, 0)),
                  pl.BlockSpec((C, C), lambda b: (0, 0)),
                  pl.BlockSpec((9, F, C), lambda b: (0, 0, 0)),
                  pl.BlockSpec((2, P), lambda b: (0, 0)),
                  pl.BlockSpec((2, P), lambda b: (0, 0))],
        out_specs=[pl.BlockSpec((1, F, P), lambda b: (b, 0, 0)),
                   pl.BlockSpec((1, F, 128), lambda b: (b, 0, 0))],
        scratch_shapes=[pltpu.VMEM((C, pe), f32),
                        pltpu.VMEM((C, pe), f32),
                        pltpu.VMEM((K, P), f32),
                        pltpu.VMEM((K, P), f32),
                        pltpu.VMEM((C, pe), bf16),
                        pltpu.VMEM((C, pe), bf16),
                        pltpu.VMEM((C, pe), bf16)],
        compiler_params=pltpu.CompilerParams(dimension_semantics=("parallel",)),
    )(x_bcp, M, w9, pmask, bmask)

    Fb = 16
    y_bn = pl.pallas_call(
        _bn_kernel,
        out_shape=jax.ShapeDtypeStruct((B, F, P), f32),
        grid=(B, F // Fb),
        in_specs=[pl.BlockSpec((1, Fb, P), lambda b, f: (b, f, 0)),
                  pl.BlockSpec((B, Fb, 128), lambda b, f: (0, f, 0)),
                  pl.BlockSpec((Fb, 1), lambda b, f: (f, 0)),
                  pl.BlockSpec((Fb, 1), lambda b, f: (f, 0))],
        out_specs=pl.BlockSpec((1, Fb, P), lambda b, f: (b, f, 0)),
        compiler_params=pltpu.CompilerParams(
            dimension_semantics=("parallel", "parallel")),
    )(y, par, bn_gamma.reshape(F, 1).astype(f32), bn_beta.reshape(F, 1).astype(f32))

    return y_bn.reshape(B, F, H, Wd)


# speedup vs baseline: 2.8856x; 1.5576x over previous
import functools

import jax
import jax.numpy as jnp
from jax import lax
from jax.experimental import pallas as pl
from jax.experimental.pallas import tpu as pltpu


def _fused_kernel(x_ref, m_ref, w_ref, pmask_ref, bmask_ref,
                  y_ref, par_ref,
                  xpad_ref, hpad_ref, hi16_ref, oh_ref,
                  f0_ref, fl_ref, fr_ref,
                  *, block_num, width, chunk):
    C = x_ref.shape[1]
    P = x_ref.shape[2] * x_ref.shape[3]
    F = y_ref.shape[1]
    K = block_num
    W = width
    f32 = jnp.float32
    bf16 = jnp.bfloat16
    pad = W

    xpad_ref[:, 0:pad] = jnp.zeros((C, pad), f32)
    xpad_ref[:, pad + P:] = jnp.zeros((C, W), f32)
    xpad_ref[:, pad:pad + P] = x_ref[0].reshape(C, P)
    x = xpad_ref[:, pad:pad + P]
    ec = pmask_ref[0:1, :]
    er = pmask_ref[1:2, :]
    xl = xpad_ref[:, pad - 1:pad - 1 + P]
    xr = xpad_ref[:, pad + 1:pad + 1 + P]
    ph = ec * xr + (1.0 - ec) * xl
    hpad_ref[:, 0:pad] = jnp.zeros((C, pad), f32)
    hpad_ref[:, pad + P:] = jnp.zeros((C, W), f32)
    hpad_ref[:, pad:pad + P] = ph
    xu = xpad_ref[:, 0:P]
    xd = xpad_ref[:, 2 * W:2 * W + P]
    pv = er * xd + (1.0 - er) * xu
    phu = hpad_ref[:, 0:P]
    phd = hpad_ref[:, 2 * W:2 * W + P]
    pd = er * phd + (1.0 - er) * phu
    hi = x - 0.25 * (x + ph + pv + pd)

    mx = jnp.max(hi, axis=0, keepdims=True)
    hi16_ref[...] = hi[0:K]
    pref = jnp.full((1, P), -jnp.inf, f32)
    for c in range(K):
        hc = hi16_ref[c:c + 1, :]
        oh_ref[c:c + 1, :] = jnp.where((hc == mx) & (hc > pref), 1.0, 0.0)
        pref = jnp.maximum(pref, hc)
    onehot = oh_ref[...]

    sums = lax.dot_general(onehot, x, (((1,), (1,)), ((), ())),
                           preferred_element_type=f32)
    counts = jnp.sum(onehot, axis=1, keepdims=True)
    means = sums / (counts + (counts == 0).astype(f32))
    M = m_ref[...]
    q = jnp.dot(means, M, preferred_element_type=f32)
    g = lax.dot_general(q, means, (((1,), (1,)), ((), ())),
                        preferred_element_type=f32)
    eye = (lax.broadcasted_iota(jnp.int32, (K, K), 0) ==
           lax.broadcasted_iota(jnp.int32, (K, K), 1)).astype(f32)
    diag_col = jnp.sum(g * eye, axis=1, keepdims=True)
    diag_row = jnp.sum(g * eye, axis=0, keepdims=True)
    quad = diag_col + diag_row - 2.0 * g
    adj = jnp.exp(-quad) * (1.0 - eye)
    adjm = jnp.dot(adj, means, preferred_element_type=f32)
    feat = x + lax.dot_general(adjm, onehot, (((0,), (0,)), ((), ())),
                               preferred_element_type=f32)

    zpadh = jnp.zeros((C, pad), bf16)
    f0_ref[:, 0:pad] = zpadh
    f0_ref[:, pad + P:] = zpadh
    f0_ref[:, pad:pad + P] = feat.astype(bf16)
    fl_ref[:, 0:pad] = zpadh
    fl_ref[:, pad + P:] = zpadh
    fl_ref[:, pad:pad + P] = (f0_ref[:, pad - 1:pad - 1 + P]
                              * bmask_ref[0:1, :])
    fr_ref[:, 0:pad] = zpadh
    fr_ref[:, pad + P:] = zpadh
    fr_ref[:, pad:pad + P] = (f0_ref[:, pad + 1:pad + 1 + P]
                              * bmask_ref[1:2, :])

    bsum = jnp.zeros((F, 1), f32)
    bsq = jnp.zeros((F, 1), f32)
    for off in range(0, P, chunk):
        acc = jnp.zeros((F, chunk), f32)
        t = 0
        for di in (-1, 0, 1):
            base = pad + di * W + off
            for fref in (fl_ref, f0_ref, fr_ref):
                s = fref[:, base:base + chunk]
                acc = acc + jnp.dot(w_ref[t], s,
                                    preferred_element_type=f32)
                t += 1
        y_ref[0, :, off:off + chunk] = acc.astype(bf16)
        bsum = bsum + jnp.sum(acc, axis=1, keepdims=True)
        bsq = bsq + jnp.sum(acc * acc, axis=1, keepdims=True)
    par_ref[0, :, 0:1] = bsum
    par_ref[0, :, 1:2] = bsq


def _bn_kernel(y_ref, par_ref, g_ref, b_ref, o_ref):
    B = par_ref.shape[0]
    Fb, P = y_ref.shape[1], y_ref.shape[2]
    H, W = o_ref.shape[2], o_ref.shape[3]
    n = B * P
    s = jnp.zeros((Fb, 1), jnp.float32)
    sq = jnp.zeros((Fb, 1), jnp.float32)
    for b in range(B):
        s = s + par_ref[b, :, 0:1]
        sq = sq + par_ref[b, :, 1:2]
    mean = s * (1.0 / n)
    var = sq * (1.0 / n) - mean * mean
    inv = lax.rsqrt(var + 1e-5)
    scale = inv * g_ref[...]
    shift = b_ref[...] - mean * scale
    val = y_ref[0].astype(jnp.float32) * scale + shift
    o_ref[0] = val.reshape(Fb, H, W)


def kernel(x, W, conv_w, bn_gamma, bn_beta):
    B, C, H, Wd = x.shape
    P = H * Wd
    K = 16
    F = conv_w.shape[0]
    f32 = jnp.float32
    bf16 = jnp.bfloat16
    chunk = 1024 if P % 1024 == 0 else P

    M = jnp.dot(W, W.T).astype(f32)
    w9 = conv_w.transpose(2, 3, 0, 1).reshape(9, F, C).astype(bf16)

    col = jnp.arange(P, dtype=jnp.int32) % Wd
    row = jnp.arange(P, dtype=jnp.int32) // Wd
    pmask = jnp.stack([(col % 2 == 0), (row % 2 == 0)]).astype(f32)
    bmask = jnp.stack([(col != 0), (col != Wd - 1)]).astype(bf16)

    kb = functools.partial(_fused_kernel, block_num=K, width=Wd, chunk=chunk)
    pe = P + 2 * Wd
    y, par = pl.pallas_call(
        kb,
        out_shape=[jax.ShapeDtypeStruct((B, F, P), bf16),
                   jax.ShapeDtypeStruct((B, F, 128), f32)],
        grid=(B,),
        in_specs=[pl.BlockSpec((1, C, H, Wd), lambda b: (b, 0, 0, 0)),
                  pl.BlockSpec((C, C), lambda b: (0, 0)),
                  pl.BlockSpec((9, F, C), lambda b: (0, 0, 0)),
                  pl.BlockSpec((2, P), lambda b: (0, 0)),
                  pl.BlockSpec((2, P), lambda b: (0, 0))],
        out_specs=[pl.BlockSpec((1, F, P), lambda b: (b, 0, 0)),
                   pl.BlockSpec((1, F, 128), lambda b: (b, 0, 0))],
        scratch_shapes=[pltpu.VMEM((C, pe), f32),
                        pltpu.VMEM((C, pe), f32),
                        pltpu.VMEM((K, P), f32),
                        pltpu.VMEM((K, P), f32),
                        pltpu.VMEM((C, pe), bf16),
                        pltpu.VMEM((C, pe), bf16),
                        pltpu.VMEM((C, pe), bf16)],
        compiler_params=pltpu.CompilerParams(dimension_semantics=("parallel",)),
    )(x, M, w9, pmask, bmask)

    Fb = 16
    y_bn = pl.pallas_call(
        _bn_kernel,
        out_shape=jax.ShapeDtypeStruct((B, F, H, Wd), f32),
        grid=(B, F // Fb),
        in_specs=[pl.BlockSpec((1, Fb, P), lambda b, f: (b, f, 0)),
                  pl.BlockSpec((B, Fb, 128), lambda b, f: (0, f, 0)),
                  pl.BlockSpec((Fb, 1), lambda b, f: (f, 0)),
                  pl.BlockSpec((Fb, 1), lambda b, f: (f, 0))],
        out_specs=pl.BlockSpec((1, Fb, H, Wd), lambda b, f: (b, f, 0, 0)),
        compiler_params=pltpu.CompilerParams(
            dimension_semantics=("parallel", "parallel")),
    )(y, par, bn_gamma.reshape(F, 1).astype(f32), bn_beta.reshape(F, 1).astype(f32))

    return y_bn


# speedup vs baseline: 3.8985x; 1.3510x over previous
import functools

import jax
import jax.numpy as jnp
from jax import lax
from jax.experimental import pallas as pl
from jax.experimental.pallas import tpu as pltpu


def _mega_kernel(x_ref, m_ref, w_ref, pmask_ref, bmask_ref, g_ref, bb_ref,
                 o_ref,
                 y_all, ps_ref, psq_ref,
                 xpad_ref, hpad_ref, hi16_ref, oh_ref,
                 f0_ref, fl_ref, fr_ref,
                 *, block_num, width, chunk):
    C = x_ref.shape[1]
    H, Wd = x_ref.shape[2], x_ref.shape[3]
    P = H * Wd
    B = y_all.shape[0]
    F = y_all.shape[1]
    K = block_num
    W = width
    f32 = jnp.float32
    bf16 = jnp.bfloat16
    pad = W

    p = pl.program_id(0)
    b = pl.program_id(1)

    @pl.when(p == 0)
    def _compute_phase():
        xpad_ref[:, 0:pad] = jnp.zeros((C, pad), f32)
        xpad_ref[:, pad + P:] = jnp.zeros((C, W), f32)
        xpad_ref[:, pad:pad + P] = x_ref[0].reshape(C, P)
        x = xpad_ref[:, pad:pad + P]

        ec = pmask_ref[0:1, :]
        er = pmask_ref[1:2, :]
        xl = xpad_ref[:, pad - 1:pad - 1 + P]
        xr = xpad_ref[:, pad + 1:pad + 1 + P]
        hsum = x + ec * xr + (1.0 - ec) * xl
        hpad_ref[:, 0:pad] = jnp.zeros((C, pad), f32)
        hpad_ref[:, pad + P:] = jnp.zeros((C, W), f32)
        hpad_ref[:, pad:pad + P] = hsum
        hu = hpad_ref[:, 0:P]
        hd = hpad_ref[:, 2 * W:2 * W + P]
        vs = er * hd + (1.0 - er) * hu
        hi = x - 0.25 * (hpad_ref[:, pad:pad + P] + vs)

        mx = jnp.max(hi, axis=0, keepdims=True)
        hi16_ref[...] = hi[0:K]
        pref = jnp.full((1, P), -jnp.inf, f32)
        for c in range(K):
            hc = hi16_ref[c:c + 1, :]
            oh_ref[c:c + 1, :] = jnp.where((hc == mx) & (hc > pref), 1.0, 0.0)
            pref = jnp.maximum(pref, hc)
        onehot = oh_ref[...]

        sums = lax.dot_general(onehot, x, (((1,), (1,)), ((), ())),
                               preferred_element_type=f32)
        counts = jnp.sum(onehot, axis=1, keepdims=True)
        means = sums / (counts + (counts == 0).astype(f32))
        M = m_ref[...]
        q = jnp.dot(means, M, preferred_element_type=f32)
        g = lax.dot_general(q, means, (((1,), (1,)), ((), ())),
                            preferred_element_type=f32)
        eye = (lax.broadcasted_iota(jnp.int32, (K, K), 0) ==
               lax.broadcasted_iota(jnp.int32, (K, K), 1)).astype(f32)
        diag_col = jnp.sum(g * eye, axis=1, keepdims=True)
        diag_row = jnp.sum(g * eye, axis=0, keepdims=True)
        quad = diag_col + diag_row - 2.0 * g
        adj = jnp.exp(-quad) * (1.0 - eye)
        adjm = jnp.dot(adj, means, preferred_element_type=f32)
        feat = x + lax.dot_general(adjm, onehot, (((0,), (0,)), ((), ())),
                                   preferred_element_type=f32)

        zpadh = jnp.zeros((C, pad), bf16)
        f0_ref[:, 0:pad] = zpadh
        f0_ref[:, pad + P:] = zpadh
        f0_ref[:, pad:pad + P] = feat.astype(bf16)
        fl_ref[:, 0:pad] = zpadh
        fl_ref[:, pad + P:] = zpadh
        fl_ref[:, pad:pad + P] = (f0_ref[:, pad - 1:pad - 1 + P]
                                  * bmask_ref[0:1, :])
        fr_ref[:, 0:pad] = zpadh
        fr_ref[:, pad + P:] = zpadh
        fr_ref[:, pad:pad + P] = (f0_ref[:, pad + 1:pad + 1 + P]
                                  * bmask_ref[1:2, :])

        bsum = jnp.zeros((F, 1), f32)
        bsq = jnp.zeros((F, 1), f32)
        for off in range(0, P, chunk):
            acc = jnp.zeros((F, chunk), f32)
            t = 0
            for di in (-1, 0, 1):
                base = pad + di * W + off
                for fref in (fl_ref, f0_ref, fr_ref):
                    s = fref[:, base:base + chunk]
                    acc = acc + jnp.dot(w_ref[t], s,
                                        preferred_element_type=f32)
                    t += 1
            y_all[b, :, off:off + chunk] = acc.astype(bf16)
            bsum = bsum + jnp.sum(acc, axis=1, keepdims=True)
            bsq = bsq + jnp.sum(acc * acc, axis=1, keepdims=True)
        zero = jnp.zeros((F, 1), f32)
        ps_ref[...] = jnp.where(b == 0, zero, ps_ref[...]) + bsum
        psq_ref[...] = jnp.where(b == 0, zero, psq_ref[...]) + bsq

    @pl.when(p == 1)
    def _bn_phase():
        n = B * P
        mean = ps_ref[...] * (1.0 / n)
        var = psq_ref[...] * (1.0 / n) - mean * mean
        inv = lax.rsqrt(var + 1e-5)
        scale = inv * g_ref[...]
        shift = bb_ref[...] - mean * scale
        rows = chunk // Wd
        for off in range(0, P, chunk):
            val = y_all[b, :, off:off + chunk].astype(f32) * scale + shift
            r0 = off // Wd
            o_ref[0, :, r0:r0 + rows, :] = val.reshape(F, rows, Wd)


def kernel(x, W, conv_w, bn_gamma, bn_beta):
    B, C, H, Wd = x.shape
    P = H * Wd
    K = 16
    F = conv_w.shape[0]
    f32 = jnp.float32
    bf16 = jnp.bfloat16
    chunk = 1024 if P % 1024 == 0 else P

    M = jnp.dot(W, W.T).astype(f32)
    w9 = conv_w.transpose(2, 3, 0, 1).reshape(9, F, C).astype(bf16)

    col = jnp.arange(P, dtype=jnp.int32) % Wd
    row = jnp.arange(P, dtype=jnp.int32) // Wd
    pmask = jnp.stack([(col % 2 == 0), (row % 2 == 0)]).astype(f32)
    bmask = jnp.stack([(col != 0), (col != Wd - 1)]).astype(bf16)

    kb = functools.partial(_mega_kernel, block_num=K, width=Wd, chunk=chunk)
    pe = P + 2 * Wd
    last = B - 1
    y_bn = pl.pallas_call(
        kb,
        out_shape=jax.ShapeDtypeStruct((B, F, H, Wd), f32),
        grid=(2, B),
        in_specs=[
            pl.BlockSpec((1, C, H, Wd),
                         lambda p, b: (b * (1 - p) + last * p, 0, 0, 0)),
            pl.BlockSpec((C, C), lambda p, b: (0, 0)),
            pl.BlockSpec((9, F, C), lambda p, b: (0, 0, 0)),
            pl.BlockSpec((2, P), lambda p, b: (0, 0)),
            pl.BlockSpec((2, P), lambda p, b: (0, 0)),
            pl.BlockSpec((F, 1), lambda p, b: (0, 0)),
            pl.BlockSpec((F, 1), lambda p, b: (0, 0)),
        ],
        out_specs=pl.BlockSpec((1, F, H, Wd), lambda p, b: (b * p, 0, 0, 0)),
        scratch_shapes=[pltpu.VMEM((B, F, P), bf16),
                        pltpu.VMEM((F, 1), f32),
                        pltpu.VMEM((F, 1), f32),
                        pltpu.VMEM((C, pe), f32),
                        pltpu.VMEM((C, pe), f32),
                        pltpu.VMEM((K, P), f32),
                        pltpu.VMEM((K, P), f32),
                        pltpu.VMEM((C, pe), bf16),
                        pltpu.VMEM((C, pe), bf16),
                        pltpu.VMEM((C, pe), bf16)],
        compiler_params=pltpu.CompilerParams(
            dimension_semantics=("arbitrary", "arbitrary")),
    )(x, M, w9, pmask, bmask,
      bn_gamma.reshape(F, 1).astype(f32), bn_beta.reshape(F, 1).astype(f32))

    return y_bn


# speedup vs baseline: 4.3667x; 1.1201x over previous
import functools

import jax
import jax.numpy as jnp
from jax import lax
from jax.experimental import pallas as pl
from jax.experimental.pallas import tpu as pltpu


def _mega_kernel(x_ref, m_ref, w_ref, pmask_ref, bmask_ref, g_ref, bb_ref,
                 o_ref,
                 y_all, ps_ref, psq_ref,
                 xpad_ref, hpad_ref, hi16_ref, oh_ref, fb_ref,
                 *, block_num, width, chunk):
    C = x_ref.shape[1]
    H, Wd = x_ref.shape[2], x_ref.shape[3]
    P = H * Wd
    B = y_all.shape[0]
    F = y_all.shape[1]
    K = block_num
    W = width
    f32 = jnp.float32
    bf16 = jnp.bfloat16
    pad = W

    p = pl.program_id(0)
    b = pl.program_id(1)

    @pl.when(p == 0)
    def _compute_phase():
        xpad_ref[:, 0:pad] = jnp.zeros((C, pad), f32)
        xpad_ref[:, pad + P:] = jnp.zeros((C, W), f32)
        xpad_ref[:, pad:pad + P] = x_ref[0].reshape(C, P)
        x = xpad_ref[:, pad:pad + P]

        ec = pmask_ref[0:1, :]
        er = pmask_ref[1:2, :]
        xl = xpad_ref[:, pad - 1:pad - 1 + P]
        xr = xpad_ref[:, pad + 1:pad + 1 + P]
        hsum = x + ec * xr + (1.0 - ec) * xl
        hpad_ref[:, 0:pad] = jnp.zeros((C, pad), f32)
        hpad_ref[:, pad + P:] = jnp.zeros((C, W), f32)
        hpad_ref[:, pad:pad + P] = hsum
        hu = hpad_ref[:, 0:P]
        hd = hpad_ref[:, 2 * W:2 * W + P]
        vs = er * hd + (1.0 - er) * hu
        hi = x - 0.25 * (hpad_ref[:, pad:pad + P] + vs)

        mx = jnp.max(hi, axis=0, keepdims=True)
        hi16_ref[...] = hi[0:K]
        pref = jnp.full((1, P), -jnp.inf, f32)
        for c in range(K):
            hc = hi16_ref[c:c + 1, :]
            oh_ref[c:c + 1, :] = jnp.where((hc == mx) & (hc > pref), 1.0, 0.0)
            pref = jnp.maximum(pref, hc)
        onehot = oh_ref[...]

        sums = lax.dot_general(onehot, x, (((1,), (1,)), ((), ())),
                               preferred_element_type=f32)
        counts = jnp.sum(onehot, axis=1, keepdims=True)
        means = sums / (counts + (counts == 0).astype(f32))
        M = m_ref[...]
        q = jnp.dot(means, M, preferred_element_type=f32)
        g = lax.dot_general(q, means, (((1,), (1,)), ((), ())),
                            preferred_element_type=f32)
        eye = (lax.broadcasted_iota(jnp.int32, (K, K), 0) ==
               lax.broadcasted_iota(jnp.int32, (K, K), 1)).astype(f32)
        diag_col = jnp.sum(g * eye, axis=1, keepdims=True)
        diag_row = jnp.sum(g * eye, axis=0, keepdims=True)
        quad = diag_col + diag_row - 2.0 * g
        adj = jnp.exp(-quad) * (1.0 - eye)
        adjm = jnp.dot(adj, means, preferred_element_type=f32)
        feat = x + lax.dot_general(adjm, onehot, (((0,), (0,)), ((), ())),
                                   preferred_element_type=f32)

        zpadh = jnp.zeros((C, pad), bf16)
        fb_ref[C:2 * C, 0:pad] = zpadh
        fb_ref[C:2 * C, pad + P:] = zpadh
        fb_ref[C:2 * C, pad:pad + P] = feat.astype(bf16)
        fb_ref[0:C, 0:pad] = zpadh
        fb_ref[0:C, pad + P:] = zpadh
        fb_ref[0:C, pad:pad + P] = (fb_ref[C:2 * C, pad - 1:pad - 1 + P]
                                    * bmask_ref[0:1, :])
        fb_ref[2 * C:, 0:pad] = zpadh
        fb_ref[2 * C:, pad + P:] = zpadh
        fb_ref[2 * C:, pad:pad + P] = (fb_ref[C:2 * C, pad + 1:pad + 1 + P]
                                       * bmask_ref[1:2, :])

        bsum = jnp.zeros((F, 1), f32)
        bsq = jnp.zeros((F, 1), f32)
        for off in range(0, P, chunk):
            acc = jnp.zeros((F, chunk), f32)
            for t, di in enumerate((-1, 0, 1)):
                base = pad + di * W + off
                s = fb_ref[:, base:base + chunk]
                acc = acc + jnp.dot(w_ref[t], s,
                                    preferred_element_type=f32)
            y_all[b, :, off:off + chunk] = acc.astype(bf16)
            bsum = bsum + jnp.sum(acc, axis=1, keepdims=True)
            bsq = bsq + jnp.sum(acc * acc, axis=1, keepdims=True)
        zero = jnp.zeros((F, 1), f32)
        ps_ref[...] = jnp.where(b == 0, zero, ps_ref[...]) + bsum
        psq_ref[...] = jnp.where(b == 0, zero, psq_ref[...]) + bsq

    @pl.when(p == 1)
    def _bn_phase():
        n = B * P
        mean = ps_ref[...] * (1.0 / n)
        var = psq_ref[...] * (1.0 / n) - mean * mean
        inv = lax.rsqrt(var + 1e-5)
        scale = inv * g_ref[...]
        shift = bb_ref[...] - mean * scale
        rows = chunk // Wd
        for off in range(0, P, chunk):
            val = y_all[b, :, off:off + chunk].astype(f32) * scale + shift
            r0 = off // Wd
            o_ref[0, :, r0:r0 + rows, :] = val.reshape(F, rows, Wd)


def kernel(x, W, conv_w, bn_gamma, bn_beta):
    B, C, H, Wd = x.shape
    P = H * Wd
    K = 16
    F = conv_w.shape[0]
    f32 = jnp.float32
    bf16 = jnp.bfloat16
    chunk = 1024 if P % 1024 == 0 else P

    M = jnp.dot(W, W.T).astype(f32)
    w9 = (conv_w.transpose(2, 3, 0, 1).reshape(3, 3, F, C)
          .transpose(0, 2, 1, 3).reshape(3, F, 3 * C).astype(bf16))

    col = jnp.arange(P, dtype=jnp.int32) % Wd
    row = jnp.arange(P, dtype=jnp.int32) // Wd
    pmask = jnp.stack([(col % 2 == 0), (row % 2 == 0)]).astype(f32)
    bmask = jnp.stack([(col != 0), (col != Wd - 1)]).astype(bf16)

    kb = functools.partial(_mega_kernel, block_num=K, width=Wd, chunk=chunk)
    pe = P + 2 * Wd
    last = B - 1
    y_bn = pl.pallas_call(
        kb,
        out_shape=jax.ShapeDtypeStruct((B, F, H, Wd), f32),
        grid=(2, B),
        in_specs=[
            pl.BlockSpec((1, C, H, Wd),
                         lambda p, b: (b * (1 - p) + last * p, 0, 0, 0)),
            pl.BlockSpec((C, C), lambda p, b: (0, 0)),
            pl.BlockSpec((3, F, 3 * C), lambda p, b: (0, 0, 0)),
            pl.BlockSpec((2, P), lambda p, b: (0, 0)),
            pl.BlockSpec((2, P), lambda p, b: (0, 0)),
            pl.BlockSpec((F, 1), lambda p, b: (0, 0)),
            pl.BlockSpec((F, 1), lambda p, b: (0, 0)),
        ],
        out_specs=pl.BlockSpec((1, F, H, Wd), lambda p, b: (b * p, 0, 0, 0)),
        scratch_shapes=[pltpu.VMEM((B, F, P), bf16),
                        pltpu.VMEM((F, 1), f32),
                        pltpu.VMEM((F, 1), f32),
                        pltpu.VMEM((C, pe), f32),
                        pltpu.VMEM((C, pe), f32),
                        pltpu.VMEM((K, P), f32),
                        pltpu.VMEM((K, P), f32),
                        pltpu.VMEM((3 * C, pe), bf16)],
        compiler_params=pltpu.CompilerParams(
            dimension_semantics=("arbitrary", "arbitrary")),
    )(x, M, w9, pmask, bmask,
      bn_gamma.reshape(F, 1).astype(f32), bn_beta.reshape(F, 1).astype(f32))

    return y_bn
